# baseline (device time: 190325 ns/iter reference)
import jax
import jax.numpy as jnp
from jax import lax
from jax.experimental import pallas as pl
from jax.experimental.pallas import tpu as pltpu

M = 2048
N = 2048
F_CHUNK = 2048

BANDS = (
    (0, 768, ("x", "y", "z")),
    (768, 768, ("y", "z", "x")),
    (1536, 512, ("z", "x", "y")),
)
R1 = M // 2
R2 = M // 4


def kernel(dy, W):
    r = lax.axis_index("x") * 2 + lax.axis_index("z")
    dy_c = lax.dynamic_slice_in_dim(dy, r * F_CHUNK, F_CHUNK, axis=1)
    w_c = lax.dynamic_slice_in_dim(W, r * F_CHUNK, F_CHUNK, axis=1)

    def body(dy_ref, w_ref, out_ref, *scratch_and_sems):
        s0 = scratch_and_sems[0:3]
        s1 = scratch_and_sems[3:6]
        s2 = scratch_and_sems[6:9]
        send_sems, recv_sems, credit_sems = scratch_and_sems[9:12]

        x = lax.axis_index("x")
        y = lax.axis_index("y")
        z = lax.axis_index("z")
        coord = {"x": x, "y": y, "z": z}

        def peer_of(axis):
            return (
                1 - x if axis == "x" else x,
                1 - y if axis == "y" else y,
                1 - z if axis == "z" else z,
            )

        barrier_sem = pltpu.get_barrier_semaphore()
        for axis in ("x", "y", "z"):
            pl.semaphore_signal(
                barrier_sem, inc=1,
                device_id=peer_of(axis), device_id_type=pl.DeviceIdType.MESH,
            )
        pl.semaphore_wait(barrier_sem, 3)

        plans = []
        for c0, C, order in BANDS:
            ch = C // 2
            a0, a1, a2 = order
            kc = c0 + coord[a0] * ch
            sc = c0 + (1 - coord[a0]) * ch
            kr1 = coord[a1] * R1
            sr1 = (1 - coord[a1]) * R1
            kr2 = kr1 + coord[a2] * R2
            sr2 = kr1 + (1 - coord[a2]) * R2
            plans.append(dict(
                ch=ch, order=order, kc=kc, sc=sc,
                kr1=kr1, sr1=sr1, kr2=kr2, sr2=sr2,
            ))

        def start(src, dst, b, slot, axis):
            rdma = pltpu.make_async_remote_copy(
                src_ref=src, dst_ref=dst,
                send_sem=send_sems.at[b * 10 + slot],
                recv_sem=recv_sems.at[b * 10 + slot],
                device_id=peer_of(axis), device_id_type=pl.DeviceIdType.MESH,
            )
            rdma.start()
            return rdma

        def gemm_cols(c, w_cols):
            out_ref[:, pl.ds(c, w_cols)] = lax.dot_general(
                dy_ref[...], w_ref[pl.ds(c, w_cols), :],
                dimension_numbers=(((1,), (1,)), ((), ())),
                preferred_element_type=jnp.float32,
            )

        rs0a = []
        for b, p in enumerate(plans):
            gemm_cols(p["sc"], p["ch"])
            rs0a.append(
                start(out_ref.at[pl.ds(p["sr1"], R1), pl.ds(p["sc"], p["ch"])],
                      s0[b], b, 0, p["order"][0]))
        for p in plans:
            gemm_cols(p["kc"], p["ch"])

        rs1, rs0b = [], []
        for b, p in enumerate(plans):
            ch, kc, sc, sr1, kr1 = (
                p["ch"], p["kc"], p["sc"], p["sr1"], p["kr1"])
            rs0a[b].wait()
            out_ref[pl.ds(sr1, R1), pl.ds(kc, ch)] = (
                out_ref[pl.ds(sr1, R1), pl.ds(kc, ch)] + s0[b][...]
            )
            rs1.append(start(out_ref.at[pl.ds(sr1, R1), pl.ds(kc, ch)],
                             s1[b], b, 1, p["order"][1]))
            pl.semaphore_signal(
                credit_sems.at[b], inc=1,
                device_id=peer_of(p["order"][0]),
                device_id_type=pl.DeviceIdType.MESH,
            )
            pl.semaphore_wait(credit_sems.at[b], 1)
            rs0b.append(
                start(out_ref.at[pl.ds(kr1, R1), pl.ds(sc, ch)],
                      s0[b], b, 2, p["order"][0]))

        for b, p in enumerate(plans):
            ch, kc, kr1 = p["ch"], p["kc"], p["kr1"]
            rs0b[b].wait()
            out_ref[pl.ds(kr1, R1), pl.ds(kc, ch)] = (
                out_ref[pl.ds(kr1, R1), pl.ds(kc, ch)] + s0[b][...]
            )

        rs2 = []
        for b, p in enumerate(plans):
            ch, kc, kr1, kr2, sr2 = (
                p["ch"], p["kc"], p["kr1"], p["kr2"], p["sr2"])
            rs1[b].wait()
            out_ref[pl.ds(sr2, R2), pl.ds(kc, ch)] = (
                out_ref[pl.ds(sr2, R2), pl.ds(kc, ch)]
                + s1[b][pl.ds(sr2 - kr1, R2), :]
            )
            rs2.append(start(out_ref.at[pl.ds(sr2, R2), pl.ds(kc, ch)],
                             s2[b], b, 3, p["order"][2]))
            out_ref[pl.ds(kr2, R2), pl.ds(kc, ch)] = (
                out_ref[pl.ds(kr2, R2), pl.ds(kc, ch)]
                + s1[b][pl.ds(kr2 - kr1, R2), :]
            )

        ag2, ag1a = [], []
        for b, p in enumerate(plans):
            ch, kc, kr2 = p["ch"], p["kc"], p["kr2"]
            rs2[b].wait()
            out_ref[pl.ds(kr2, R2), pl.ds(kc, ch)] = (
                out_ref[pl.ds(kr2, R2), pl.ds(kc, ch)] + s2[b][...]
            )
            src = out_ref.at[pl.ds(kr2, R2), pl.ds(kc, ch)]
            ag2.append(start(src, src, b, 4, p["order"][2]))
            ag1a.append(start(src, src, b, 5, p["order"][1]))

        ag1b, ag0a = [], []
        for b, p in enumerate(plans):
            ch, kc, kr1, sr2 = p["ch"], p["kc"], p["kr1"], p["sr2"]
            ag2[b].wait()
            srcb = out_ref.at[pl.ds(sr2, R2), pl.ds(kc, ch)]
            ag1b.append(start(srcb, srcb, b, 6, p["order"][1]))
            srca = out_ref.at[pl.ds(kr1, R1), pl.ds(kc, ch)]
            ag0a.append(start(srca, srca, b, 7, p["order"][0]))

        ag0b, ag0c = [], []
        for b, p in enumerate(plans):
            ch, kc, sr1 = p["ch"], p["kc"], p["sr1"]
            a2c = coord[p["order"][2]]
            ag1a[b].wait()
            rb = sr1 + a2c * R2
            src = out_ref.at[pl.ds(rb, R2), pl.ds(kc, ch)]
            ag0b.append(start(src, src, b, 8, p["order"][0]))
        for b, p in enumerate(plans):
            ch, kc, sr1 = p["ch"], p["kc"], p["sr1"]
            a2c = coord[p["order"][2]]
            ag1b[b].wait()
            rc = sr1 + (1 - a2c) * R2
            src = out_ref.at[pl.ds(rc, R2), pl.ds(kc, ch)]
            ag0c.append(start(src, src, b, 9, p["order"][0]))

        for rdma in ag0a + ag0b + ag0c:
            rdma.wait()

    scratch_shapes = (
        [pltpu.VMEM((R1, C // 2), jnp.float32) for _, C, _ in BANDS]
        + [pltpu.VMEM((R1, C // 2), jnp.float32) for _, C, _ in BANDS]
        + [pltpu.VMEM((R2, C // 2), jnp.float32) for _, C, _ in BANDS]
        + [pltpu.SemaphoreType.DMA((30,)), pltpu.SemaphoreType.DMA((30,)),
           pltpu.SemaphoreType.REGULAR((len(BANDS),))]
    )
    return pl.pallas_call(
        body,
        out_shape=jax.ShapeDtypeStruct((M, N), jnp.float32),
        in_specs=[
            pl.BlockSpec(memory_space=pltpu.VMEM),
            pl.BlockSpec(memory_space=pltpu.VMEM),
        ],
        out_specs=pl.BlockSpec(memory_space=pltpu.VMEM),
        scratch_shapes=scratch_shapes,
        compiler_params=pltpu.CompilerParams(
            collective_id=0,
            vmem_limit_bytes=63 * 1024 * 1024,
        ),
    )(dy_c, w_c)


# device time: 175901 ns/iter; 1.0820x vs baseline; 1.0820x over previous
import jax
import jax.numpy as jnp
from jax import lax
from jax.experimental import pallas as pl
from jax.experimental.pallas import tpu as pltpu

M = 2048
N = 2048
F_CHUNK = 2048

_O = (("x", "y", "z"), ("y", "z", "x"), ("z", "x", "y"))
_SIZES = (384, 320, 320, 384, 320, 320)
GROUPS = tuple(
    (sum(_SIZES[:g]), s, _O[g % 3]) for g, s in enumerate(_SIZES)
)
SCRATCH_ROWS = sum(s // 2 + s // 4 + s // 8 for _, s, _ in GROUPS)


def kernel(dy, W):
    r = lax.axis_index("x") * 2 + lax.axis_index("z")
    dy_c = lax.dynamic_slice_in_dim(dy, r * F_CHUNK, F_CHUNK, axis=1)
    w_c = lax.dynamic_slice_in_dim(W, r * F_CHUNK, F_CHUNK, axis=1)

    def body(dy_ref, w_ref, out_ref, scratch, send_sems, recv_sems):
        x = lax.axis_index("x")
        y = lax.axis_index("y")
        z = lax.axis_index("z")
        coord = {"x": x, "y": y, "z": z}

        def peer_of(axis):
            return (
                1 - x if axis == "x" else x,
                1 - y if axis == "y" else y,
                1 - z if axis == "z" else z,
            )

        barrier_sem = pltpu.get_barrier_semaphore()
        for axis in ("x", "y", "z"):
            pl.semaphore_signal(
                barrier_sem, inc=1,
                device_id=peer_of(axis), device_id_type=pl.DeviceIdType.MESH,
            )
        pl.semaphore_wait(barrier_sem, 3)

        plans = []
        soff = 0
        for g0, rows, order in GROUPS:
            keep = g0
            phases = []
            for ph, axis in enumerate(order):
                h = rows >> (ph + 1)
                k = keep + coord[axis] * h
                snd = keep + (1 - coord[axis]) * h
                phases.append((axis, h, k, snd, soff))
                keep = k
                soff += h
            plans.append(phases)

        def start(src, dst, sem_idx, axis):
            rdma = pltpu.make_async_remote_copy(
                src_ref=src, dst_ref=dst,
                send_sem=send_sems.at[sem_idx], recv_sem=recv_sems.at[sem_idx],
                device_id=peer_of(axis), device_id_type=pl.DeviceIdType.MESH,
            )
            rdma.start()
            return rdma

        def start_rs(g, ph):
            axis, h, _k, snd, so = plans[g][ph]
            return start(out_ref.at[pl.ds(snd, h)], scratch.at[pl.ds(so, h)],
                         g * 3 + ph, axis)

        def start_ag(g, ph):
            axis, h, k, _snd, _so = plans[g][ph]
            return start(out_ref.at[pl.ds(k, h)], out_ref.at[pl.ds(k, h)],
                         (len(GROUPS) + g) * 3 + ph, axis)

        def gemm(off, h):
            return lax.dot_general(
                dy_ref[pl.ds(off, h), :], w_ref[...],
                dimension_numbers=(((1,), (1,)), ((), ())),
                preferred_element_type=jnp.float32,
            )

        rdmas = []
        for g in range(len(GROUPS)):
            _axis, h, _k, snd, _so = plans[g][0]
            out_ref[pl.ds(snd, h), :] = gemm(snd, h)
            rdmas.append(start_rs(g, 0))

        nxt = []
        for g in range(len(GROUPS)):
            _axis, h, k, _snd, so = plans[g][0]
            rdmas[g].wait()
            out_ref[pl.ds(k, h), :] = gemm(k, h) + scratch[pl.ds(so, h), :]
            nxt.append(start_rs(g, 1))
        rdmas = nxt

        for ph in (1, 2):
            nxt = []
            for g in range(len(GROUPS)):
                _axis, h, k, _snd, so = plans[g][ph]
                rdmas[g].wait()
                out_ref[pl.ds(k, h), :] = (
                    out_ref[pl.ds(k, h), :] + scratch[pl.ds(so, h), :]
                )
                nxt.append(start_rs(g, ph + 1) if ph < 2 else start_ag(g, 2))
            rdmas = nxt

        for ph in (1, 0):
            nxt = []
            for g in range(len(GROUPS)):
                rdmas[g].wait()
                nxt.append(start_ag(g, ph))
            rdmas = nxt
        for rdma in rdmas:
            rdma.wait()

    return pl.pallas_call(
        body,
        out_shape=jax.ShapeDtypeStruct((M, N), jnp.float32),
        in_specs=[
            pl.BlockSpec(memory_space=pltpu.VMEM),
            pl.BlockSpec(memory_space=pltpu.VMEM),
        ],
        out_specs=pl.BlockSpec(memory_space=pltpu.VMEM),
        scratch_shapes=[
            pltpu.VMEM((SCRATCH_ROWS, N), jnp.float32),
            pltpu.SemaphoreType.DMA((len(GROUPS) * 6,)),
            pltpu.SemaphoreType.DMA((len(GROUPS) * 6,)),
        ],
        compiler_params=pltpu.CompilerParams(
            collective_id=0,
            vmem_limit_bytes=63 * 1024 * 1024,
        ),
    )(dy_c, w_c)
